# baseline (speedup 1.0000x reference)
import functools

import jax
import jax.numpy as jnp
from jax import lax
from jax.experimental import pallas as pl
from jax.experimental.pallas import tpu as pltpu
from jax.experimental.pallas import tpu_sc as plsc


def _make_sc_broadcast(B: int, S: int, D: int, dtype):
    info = plsc.get_sparse_core_info()
    NC, NS = info.num_cores, info.num_subcores
    NW = NC * NS
    assert S % NW == 0
    rows_per_w = S // NW
    buf_rows = min(rows_per_w, max(8, (131071 // D // 2) & ~7))
    n_full, rem = divmod(rows_per_w, buf_rows)
    chunks = [buf_rows] * n_full + ([rem] if rem else [])
    offs = [i * buf_rows for i in range(len(chunks))]

    mesh = plsc.VectorSubcoreMesh(core_axis_name="c", subcore_axis_name="s")

    @functools.partial(
        pl.kernel,
        mesh=mesh,
        out_type=jax.ShapeDtypeStruct((B, S, D), dtype),
        scratch_types=[
            pltpu.VMEM((buf_rows, D), dtype),
            pltpu.VMEM((buf_rows, D), dtype),
            pltpu.SemaphoreType.DMA,
            pltpu.SemaphoreType.DMA,
            pltpu.SemaphoreType.DMA,
        ],
    )
    def broadcast_rows(table_hbm, out_hbm, buf0, buf1, rs0, rs1, wsem):
        bufs, rsems = (buf0, buf1), (rs0, rs1)
        wid = lax.axis_index("s") * NC + lax.axis_index("c")
        base = wid * rows_per_w

        def start_read(j):
            r0 = base + offs[j]
            c = chunks[j]
            dst = bufs[j % 2] if c == buf_rows else bufs[j % 2].at[pl.ds(0, c), :]
            cp = pltpu.make_async_copy(
                table_hbm.at[pl.ds(r0, c), :], dst, rsems[j % 2])
            cp.start()
            return cp

        rd = start_read(0)
        for j in range(len(chunks)):
            p = j % 2
            c = chunks[j]
            rd.wait()
            if j + 1 < len(chunks):
                rd = start_read(j + 1)
            r0 = base + offs[j]
            src = bufs[p] if c == buf_rows else bufs[p].at[pl.ds(0, c), :]
            cps = []
            for b in range(B):
                cp = pltpu.make_async_copy(
                    src, out_hbm.at[b, pl.ds(r0, c), :], wsem)
                cp.start()
                cps.append(cp)
            for cp in cps:
                cp.wait()

    return broadcast_rows


def kernel(x, position_embedding):
    B, S, _ = x.shape
    _, D = position_embedding.shape
    fn = _make_sc_broadcast(B, S, D, position_embedding.dtype)
    return fn(position_embedding)
